# baseline (speedup 1.0000x reference)
import functools

import jax
import jax.numpy as jnp
from jax import lax
from jax.experimental import pallas as pl
from jax.experimental.pallas import tpu as pltpu
from jax.experimental.pallas import tpu_sc as plsc

N = 10000
E = 320000
NC = 2
NS = 16
NW = NC * NS
EPW = E // NW
NPAD = 10240
RPT = NPAD // NS

_CFG = {64: (125, 80, 8), 16: (125, 80, 8)}


def _mesh():
  return plsc.VectorSubcoreMesh(
      core_axis_name="c", subcore_axis_name="s", num_cores=NC, num_subcores=NS)


@functools.lru_cache(maxsize=None)
def _make_prop(F):
  CH, NCHUNK, U = _CFG[F]
  NG = NCHUNK // U

  @functools.partial(
      pl.kernel,
      out_type=jax.ShapeDtypeStruct((NC, NPAD, F), jnp.float32),
      mesh=_mesh(),
      compiler_params=pltpu.CompilerParams(use_tc_tiling_on_sc=False),
      scratch_types=[
          pltpu.VMEM((NCHUNK, CH), jnp.int32),
          pltpu.VMEM((NCHUNK, CH), jnp.int32),
          pltpu.VMEM((U, CH, F), jnp.float32),
          pltpu.VMEM_SHARED((NPAD, F), jnp.float32),
          pltpu.SemaphoreType.DMA((U,)),
          pltpu.SemaphoreType.DMA((U,)),
      ],
  )
  def prop(w_hbm, src_hbm, dst_hbm, zeros_hbm, out_hbm,
           idx_s, idx_d, rows, acc, gsem, ssem):
    c = lax.axis_index("c")
    s = lax.axis_index("s")
    wid = c * NS + s
    cps = [
        pltpu.async_copy(src_hbm.at[wid], idx_s, gsem.at[0]),
        pltpu.async_copy(dst_hbm.at[wid], idx_d, gsem.at[U - 1]),
        pltpu.async_copy(zeros_hbm.at[pl.ds(s * RPT, RPT)],
                         acc.at[pl.ds(s * RPT, RPT)], ssem.at[0]),
    ]
    for cp in cps:
      cp.wait()
    plsc.subcore_barrier()

    def body(i, carry):
      base = i * U
      for j in range(U):
        @pl.when(i > 0)
        def _drain(j=j):
          pltpu.make_async_copy(
              rows.at[j], acc.at[idx_d.at[base - U + j]], ssem.at[j]).wait()
        pltpu.async_copy(w_hbm.at[idx_s.at[base + j]], rows.at[j],
                         gsem.at[j])
      for j in range(U):
        pltpu.make_async_copy(w_hbm.at[idx_s.at[base + j]], rows.at[j],
                              gsem.at[j]).wait()
        pltpu.async_copy(rows.at[j], acc.at[idx_d.at[base + j]],
                         ssem.at[j], add=True)
      return carry

    lax.fori_loop(0, NG, body, 0)
    for j in range(U):
      pltpu.make_async_copy(
          rows.at[j], acc.at[idx_d.at[(NG - 1) * U + j]], ssem.at[j]).wait()
    plsc.subcore_barrier()
    pltpu.sync_copy(acc.at[pl.ds(s * RPT, RPT)],
                    out_hbm.at[c, pl.ds(s * RPT, RPT)])

  return prop


@functools.lru_cache(maxsize=None)
def _make_prop2():
  F = 64
  CH, NCHUNK, U = _CFG[F]
  NG = NCHUNK // U

  @functools.partial(
      pl.kernel,
      out_type=jax.ShapeDtypeStruct((2, NC, NPAD, F), jnp.float32),
      mesh=_mesh(),
      compiler_params=pltpu.CompilerParams(use_tc_tiling_on_sc=False),
      scratch_types=[
          pltpu.VMEM((NCHUNK, CH), jnp.int32),
          pltpu.VMEM((NCHUNK, CH), jnp.int32),
          pltpu.VMEM((U, CH, F), jnp.float32),
          pltpu.VMEM_SHARED((NPAD, F), jnp.float32),
          pltpu.SemaphoreType.DMA((U,)),
          pltpu.SemaphoreType.DMA((U,)),
      ],
  )
  def prop2(wl_hbm, wr_hbm, src_hbm, dst_hbm, zeros_hbm, out_hbm,
            idx_s, idx_d, rows, acc, gsem, ssem):
    c = lax.axis_index("c")
    s = lax.axis_index("s")
    wid = c * NS + s
    cps = [
        pltpu.async_copy(src_hbm.at[wid], idx_s, gsem.at[0]),
        pltpu.async_copy(dst_hbm.at[wid], idx_d, gsem.at[U - 1]),
        pltpu.async_copy(zeros_hbm.at[pl.ds(s * RPT, RPT)],
                         acc.at[pl.ds(s * RPT, RPT)], ssem.at[0]),
    ]
    for cp in cps:
      cp.wait()
    plsc.subcore_barrier()

    def run_pass(w_hbm):
      def body(i, carry):
        base = i * U
        for j in range(U):
          @pl.when(i > 0)
          def _drain(j=j):
            pltpu.make_async_copy(
                rows.at[j], acc.at[idx_d.at[base - U + j]], ssem.at[j]).wait()
          pltpu.async_copy(w_hbm.at[idx_s.at[base + j]], rows.at[j],
                           gsem.at[j])
        for j in range(U):
          pltpu.make_async_copy(w_hbm.at[idx_s.at[base + j]], rows.at[j],
                                gsem.at[j]).wait()
          pltpu.async_copy(rows.at[j], acc.at[idx_d.at[base + j]],
                           ssem.at[j], add=True)
        return carry

      lax.fori_loop(0, NG, body, 0)
      for j in range(U):
        pltpu.make_async_copy(
            rows.at[j], acc.at[idx_d.at[(NG - 1) * U + j]], ssem.at[j]).wait()
      plsc.subcore_barrier()

    run_pass(wl_hbm)
    pltpu.sync_copy(acc.at[pl.ds(s * RPT, RPT)],
                    out_hbm.at[0, c, pl.ds(s * RPT, RPT)])
    pltpu.sync_copy(zeros_hbm.at[pl.ds(s * RPT, RPT)],
                    acc.at[pl.ds(s * RPT, RPT)])
    plsc.subcore_barrier()
    run_pass(wr_hbm)
    pltpu.sync_copy(acc.at[pl.ds(s * RPT, RPT)],
                    out_hbm.at[1, c, pl.ds(s * RPT, RPT)])

  return prop2


L2CH = 100
L2NCHUNK = 200
L2U = 8
L2NG = L2NCHUNK // L2U


@functools.lru_cache(maxsize=None)
def _make_l2chain():
  @functools.partial(
      pl.kernel,
      out_type=[
          jax.ShapeDtypeStruct((NC, NPAD, 2, 16), jnp.float32),
          jax.ShapeDtypeStruct((NC, NPAD, 2, 16), jnp.float32),
      ],
      mesh=_mesh(),
      compiler_params=pltpu.CompilerParams(use_tc_tiling_on_sc=False),
      scratch_types=[
          pltpu.VMEM((L2NCHUNK, L2CH), jnp.int32),
          pltpu.VMEM((L2NCHUNK, L2CH), jnp.int32),
          pltpu.VMEM((L2U, L2CH, 2, 16), jnp.float32),
          pltpu.VMEM_SHARED((NPAD, 2, 16), jnp.float32),
          pltpu.VMEM((RPT, 2, 16), jnp.float32),
          pltpu.VMEM((RPT, 2, 16), jnp.float32),
          pltpu.SemaphoreType.DMA((L2U,)),
          pltpu.SemaphoreType.DMA((L2U,)),
      ],
  )
  def l2chain(w0_hbm, gd2_hbm, gd1_hbm, db2_hbm, db1_hbm,
              src_hbm, dst_hbm, zeros_hbm, u_hbm, wscr_hbm,
              idx_s, idx_d, rows, acc, abuf, bbuf, gsem, ssem):
    c = lax.axis_index("c")
    s = lax.axis_index("s")
    rowsl = pl.ds(s * RPT, RPT)
    cps = [
        pltpu.async_copy(src_hbm.at[s], idx_s, gsem.at[0]),
        pltpu.async_copy(dst_hbm.at[s], idx_d, gsem.at[L2U - 1]),
        pltpu.async_copy(zeros_hbm.at[rowsl], acc.at[rowsl], ssem.at[0]),
    ]
    for cp in cps:
      cp.wait()
    plsc.subcore_barrier()

    def run_pass(w_ref):
      def body(i, carry):
        base = i * L2U
        for j in range(L2U):
          @pl.when(i > 0)
          def _drain(j=j):
            pltpu.make_async_copy(
                rows.at[j], acc.at[idx_d.at[base - L2U + j]],
                ssem.at[j]).wait()
          pltpu.async_copy(w_ref.at[idx_s.at[base + j]], rows.at[j],
                           gsem.at[j])
        for j in range(L2U):
          pltpu.make_async_copy(w_ref.at[idx_s.at[base + j]], rows.at[j],
                                gsem.at[j]).wait()
          pltpu.async_copy(rows.at[j], acc.at[idx_d.at[base + j]],
                           ssem.at[j], add=True)
        return carry

      lax.fori_loop(0, L2NG, body, 0)
      for j in range(L2U):
        pltpu.make_async_copy(
            rows.at[j], acc.at[idx_d.at[(L2NG - 1) * L2U + j]],
            ssem.at[j]).wait()
      plsc.subcore_barrier()

    def mul_into_abuf(scale_hbm):
      cps = [pltpu.async_copy(acc.at[rowsl], abuf, gsem.at[0]),
             pltpu.async_copy(scale_hbm.at[rowsl], bbuf, gsem.at[1])]
      for cp in cps:
        cp.wait()

      def body(r, carry):
        for k in range(2):
          abuf[r, k, :] = abuf[r, k, :] * bbuf[r, k, :]
        return carry

      lax.fori_loop(0, RPT, body, 0)

    def update(gd_hbm):
      mul_into_abuf(db2_hbm)
      cps = [pltpu.async_copy(gd_hbm.at[c, rowsl], bbuf, gsem.at[1]),
             pltpu.async_copy(zeros_hbm.at[rowsl], acc.at[rowsl],
                              ssem.at[0])]
      for cp in cps:
        cp.wait()

      def body(r, carry):
        for k in range(2):
          abuf[r, k, :] = abuf[r, k, :] + bbuf[r, k, :]
        return carry

      lax.fori_loop(0, RPT, body, 0)
      pltpu.sync_copy(abuf, wscr_hbm.at[c, rowsl])
      plsc.subcore_barrier()

    run_pass(w0_hbm.at[c])
    update(gd2_hbm)
    run_pass(wscr_hbm.at[c])
    update(gd1_hbm)
    run_pass(wscr_hbm.at[c])
    mul_into_abuf(db1_hbm)
    pltpu.sync_copy(abuf, u_hbm.at[c, rowsl])

  return l2chain


@functools.lru_cache(maxsize=None)
def _make_deg():
  CH, NCHUNK, U = _CFG[16]
  NG = NCHUNK // U

  @functools.partial(
      pl.kernel,
      out_type=jax.ShapeDtypeStruct((NC, NPAD, 16), jnp.float32),
      mesh=_mesh(),
      compiler_params=pltpu.CompilerParams(use_tc_tiling_on_sc=False),
      scratch_types=[
          pltpu.VMEM((NCHUNK, CH), jnp.int32),
          pltpu.VMEM((CH, 16), jnp.float32),
          pltpu.VMEM_SHARED((NPAD, 16), jnp.float32),
          pltpu.SemaphoreType.DMA((U,)),
      ],
  )
  def deg_kernel(ones_hbm, dst_hbm, zeros_hbm, out_hbm, idx_d, ones_v, acc,
                 ssem):
    c = lax.axis_index("c")
    s = lax.axis_index("s")
    wid = c * NS + s
    cps = [
        pltpu.async_copy(dst_hbm.at[wid], idx_d, ssem.at[0]),
        pltpu.async_copy(ones_hbm, ones_v, ssem.at[1]),
        pltpu.async_copy(zeros_hbm.at[pl.ds(s * RPT, RPT)],
                         acc.at[pl.ds(s * RPT, RPT)], ssem.at[2]),
    ]
    for cp in cps:
      cp.wait()
    plsc.subcore_barrier()

    def body(i, carry):
      base = i * U
      for j in range(U):
        @pl.when(i > 0)
        def _drain(j=j):
          pltpu.make_async_copy(
              ones_v, acc.at[idx_d.at[base - U + j]], ssem.at[j]).wait()
        pltpu.async_copy(ones_v, acc.at[idx_d.at[base + j]], ssem.at[j],
                         add=True)
      return carry

    lax.fori_loop(0, NG, body, 0)
    for j in range(U):
      pltpu.make_async_copy(
          ones_v, acc.at[idx_d.at[(NG - 1) * U + j]], ssem.at[j]).wait()
    plsc.subcore_barrier()
    pltpu.sync_copy(acc.at[pl.ds(s * RPT, RPT)],
                    out_hbm.at[c, pl.ds(s * RPT, RPT)])

  return deg_kernel


R = 2000
G = N // R
_P = jax.lax.Precision.HIGHEST


def _tc_call(body, in_specs, out_specs, out_shapes, g=G):
  return pl.pallas_call(
      body,
      grid=(g,),
      in_specs=in_specs,
      out_specs=out_specs,
      out_shape=out_shapes,
  )


RP = 2048


def _r2(r, cols):
  return pl.BlockSpec((r, cols), lambda i: (i, 0))


def _r3(r, cols):
  return pl.BlockSpec((NC, r, cols), lambda i: (0, i, 0))


def _r4(r):
  return pl.BlockSpec((NC, r, 2, 16), lambda i: (0, i, 0, 0))


def _r216(r):
  return pl.BlockSpec((r, 2, 16), lambda i: (i, 0, 0))


def _b2(shape):
  nd = len(shape)
  return pl.BlockSpec(shape, lambda i: (0,) * nd)


_vp64 = pl.BlockSpec((NC, R, 64), lambda i: (0, i, 0))
_vp2 = pl.BlockSpec((2, NC, R, 64), lambda i: (0, 0, i, 0))
_n128 = pl.BlockSpec((R, 128), lambda i: (i, 0))
_n64 = pl.BlockSpec((R, 64), lambda i: (i, 0))
_n16 = pl.BlockSpec((R, 16), lambda i: (i, 0))


def _prep_body(degp, x, w10, acc1, wl, wr, dinv, dinv2, db1, db2):
  deg = degp[0, :, :] + degp[1, :, :]
  di = jnp.where(deg > 0.0, lax.rsqrt(jnp.maximum(deg, 1e-30)), 0.0)
  di2 = di * di
  dinv[...] = di
  dinv2[...] = di2
  rb = di.shape[0]
  db1[...] = jnp.broadcast_to(di[:, 0:1, None], (rb, 2, 16))
  db2[...] = jnp.broadcast_to(di2[:, 0:1, None], (rb, 2, 16))
  xb = x[...]
  acc1[...] = jnp.dot(xb, w10[...], precision=_P)
  w = xb * di[:, 0:1]
  wl[...] = w[:, :64]
  wr[...] = w[:, 64:]


def _step1_body(vp2, dinv, dinv2, acc_in, wk, acc_out, wl, wr):
  v = jnp.concatenate([vp2[0, 0, :, :] + vp2[0, 1, :, :],
                       vp2[1, 0, :, :] + vp2[1, 1, :, :]], axis=1)
  di = dinv[:, 0:1]
  acc_out[...] = acc_in[...] + jnp.dot(v * di, wk[...], precision=_P)
  w = v * dinv2[:, 0:1]
  wl[...] = w[:, :64]
  wr[...] = w[:, 64:]


def _split216(a):
  rb = a.shape[0]
  return jnp.stack([a[:, :32].reshape(rb, 2, 16),
                    a[:, 32:].reshape(rb, 2, 16)], axis=0)


def _l1fin_body(vp2, dinv, acc_in, w13, b1, w20, w21, w22, w23,
                g0, gd2, gd1, w0):
  v = jnp.concatenate([vp2[0, 0, :, :] + vp2[0, 1, :, :],
                       vp2[1, 0, :, :] + vp2[1, 1, :, :]], axis=1)
  di = dinv[:, 0:1]
  h = acc_in[...] + jnp.dot(v * di, w13[...], precision=_P) + b1[...]
  h = jnp.maximum(h, 0.0)
  g0[...] = jnp.dot(h, w20[...], precision=_P)
  gd2[...] = _split216(jnp.dot(h, w22[...], precision=_P) * di)
  gd1[...] = _split216(jnp.dot(h, w21[...], precision=_P) * di)
  w0[...] = _split216(jnp.dot(h, w23[...], precision=_P) * di)


def _fin_body(u, g0, b2, out):
  rb = g0.shape[0]
  u64 = jnp.concatenate([u[0, :, :, :].reshape(rb, 32),
                         u[1, :, :, :].reshape(rb, 32)], axis=1)
  t = g0[...] + u64 + b2[...]
  t = t - jnp.max(t, axis=1, keepdims=True)
  out[...] = t - jnp.log(jnp.sum(jnp.exp(t), axis=1, keepdims=True))


def kernel(x, edge_index, W1, b1, W2, b2):
  f32 = jnp.float32
  ch2, nch2, _ = _CFG[64]
  src2 = edge_index[0].reshape(NW, nch2, ch2)
  dst2 = edge_index[1].reshape(NW, nch2, ch2)
  srcl2 = edge_index[0].reshape(NS, L2NCHUNK, L2CH)
  dstl2 = edge_index[1].reshape(NS, L2NCHUNK, L2CH)
  z64 = jnp.zeros((NPAD, 64), f32)
  z32 = jnp.zeros((NPAD, 2, 16), f32)
  z16 = jnp.zeros((NPAD, 16), f32)
  ones16 = jnp.ones((_CFG[16][0], 16), f32)
  b1r = b1.reshape(1, 128)
  b2r = b2.reshape(1, 64)

  nshape128 = jax.ShapeDtypeStruct((N, 128), f32)
  nshape64 = jax.ShapeDtypeStruct((N, 64), f32)
  nshape16 = jax.ShapeDtypeStruct((N, 16), f32)
  npad216 = jax.ShapeDtypeStruct((NPAD, 2, 16), f32)
  nc216 = jax.ShapeDtypeStruct((NC, N, 2, 16), f32)
  ncp216 = jax.ShapeDtypeStruct((NC, NPAD, 2, 16), f32)

  deg_kernel = _make_deg()
  prop2 = _make_prop2()
  l2chain = _make_l2chain()

  chd, nchd, _ = _CFG[16]
  dstd = edge_index[1].reshape(NW, nchd, chd)
  degp = deg_kernel(ones16, dstd, z16)

  acc1, wl, wr, dinv, dinv2, db1, db2 = _tc_call(
      _prep_body,
      [_r3(RP, 16), _r2(RP, 128), _b2((128, 128))],
      [_r2(RP, 128), _r2(RP, 64), _r2(RP, 64), _r2(RP, 16), _r2(RP, 16),
       _r216(RP), _r216(RP)],
      [nshape128, nshape64, nshape64, nshape16, nshape16, npad216, npad216],
      g=NPAD // RP,
  )(degp, x, W1[0])

  for k in (1, 2):
    vp2 = prop2(wl, wr, src2, dst2, z64)
    acc1, wl, wr = _tc_call(
        _step1_body,
        [_vp2, _n16, _n16, _n128, _b2((128, 128))],
        [_n128, _n64, _n64],
        [nshape128, nshape64, nshape64],
    )(vp2, dinv, dinv2, acc1, W1[k])

  vp2 = prop2(wl, wr, src2, dst2, z64)
  g0, gd2sc, gd1sc, w0sc = _tc_call(
      _l1fin_body,
      [_vp2, _n16, _n128, _b2((128, 128)), _b2((1, 128)),
       _b2((128, 64)), _b2((128, 64)), _b2((128, 64)), _b2((128, 64))],
      [_n64, _r4(R), _r4(R), _r4(R)],
      [nshape64, ncp216, ncp216, nc216],
  )(vp2, dinv, acc1, W1[3], b1r, W2[0], W2[1], W2[2], W2[3])

  u_out, _ = l2chain(w0sc, gd2sc, gd1sc, db2, db1, srcl2, dstl2, z32)

  (out,) = _tc_call(
      _fin_body,
      [_r4(R), _n64, _b2((1, 64))],
      [_n64],
      [nshape64],
  )(u_out, g0, b2r)
  return out


# speedup vs baseline: 1.2823x; 1.2823x over previous
import functools

import jax
import jax.numpy as jnp
from jax import lax
from jax.experimental import pallas as pl
from jax.experimental.pallas import tpu as pltpu
from jax.experimental.pallas import tpu_sc as plsc

N = 10000
E = 320000
NC = 2
NS = 16
NW = NC * NS
EPW = E // NW
CH = 125
NCHUNK = EPW // CH
U = 8
NG = NCHUNK // U
F = 64
NPAD = 10240
RPT = NPAD // NS


def _mesh():
  return plsc.VectorSubcoreMesh(
      core_axis_name="c", subcore_axis_name="s", num_cores=NC, num_subcores=NS)


def _prologue(src_hbm, dst_hbm, zeros_hbm, idx_s, idx_d, acc, gsem, ssem, wid,
              s):
  cps = [
      pltpu.async_copy(src_hbm.at[wid], idx_s, gsem.at[0]),
      pltpu.async_copy(dst_hbm.at[wid], idx_d, gsem.at[U - 1]),
      pltpu.async_copy(zeros_hbm.at[pl.ds(s * RPT, RPT)],
                       acc.at[pl.ds(s * RPT, RPT)], ssem.at[0]),
  ]
  for cp in cps:
    cp.wait()
  plsc.subcore_barrier()


def _pipelined_pass(w_hbm, idx_s, idx_d, rows, acc, gsem, ssem):
  def body(i, carry):
    base = i * U
    for j in range(U):
      @pl.when(i > 0)
      def _drain(j=j):
        pltpu.make_async_copy(
            rows.at[j], acc.at[idx_d.at[base - U + j]], ssem.at[j]).wait()
      pltpu.async_copy(w_hbm.at[idx_s.at[base + j]], rows.at[j], gsem.at[j])
    for j in range(U):
      pltpu.make_async_copy(w_hbm.at[idx_s.at[base + j]], rows.at[j],
                            gsem.at[j]).wait()
      pltpu.async_copy(rows.at[j], acc.at[idx_d.at[base + j]],
                       ssem.at[j], add=True)
    return carry

  lax.fori_loop(0, NG, body, 0)
  for j in range(U):
    pltpu.make_async_copy(
        rows.at[j], acc.at[idx_d.at[(NG - 1) * U + j]], ssem.at[j]).wait()
  plsc.subcore_barrier()


def _sc_scratch():
  return [
      pltpu.VMEM((NCHUNK, CH), jnp.int32),
      pltpu.VMEM((NCHUNK, CH), jnp.int32),
      pltpu.VMEM((U, CH, F), jnp.float32),
      pltpu.VMEM_SHARED((NPAD, F), jnp.float32),
      pltpu.SemaphoreType.DMA((U,)),
      pltpu.SemaphoreType.DMA((U,)),
  ]


@functools.lru_cache(maxsize=None)
def _make_prop():
  @functools.partial(
      pl.kernel,
      out_type=jax.ShapeDtypeStruct((NC, NPAD, F), jnp.float32),
      mesh=_mesh(),
      compiler_params=pltpu.CompilerParams(use_tc_tiling_on_sc=False),
      scratch_types=_sc_scratch(),
  )
  def prop(w_hbm, src_hbm, dst_hbm, zeros_hbm, out_hbm,
           idx_s, idx_d, rows, acc, gsem, ssem):
    c = lax.axis_index("c")
    s = lax.axis_index("s")
    _prologue(src_hbm, dst_hbm, zeros_hbm, idx_s, idx_d, acc, gsem, ssem,
              c * NS + s, s)
    _pipelined_pass(w_hbm, idx_s, idx_d, rows, acc, gsem, ssem)
    pltpu.sync_copy(acc.at[pl.ds(s * RPT, RPT)],
                    out_hbm.at[c, pl.ds(s * RPT, RPT)])

  return prop


@functools.lru_cache(maxsize=None)
def _make_prop2():
  @functools.partial(
      pl.kernel,
      out_type=jax.ShapeDtypeStruct((2, NC, NPAD, F), jnp.float32),
      mesh=_mesh(),
      compiler_params=pltpu.CompilerParams(use_tc_tiling_on_sc=False),
      scratch_types=_sc_scratch(),
  )
  def prop2(wl_hbm, wr_hbm, src_hbm, dst_hbm, zeros_hbm, out_hbm,
            idx_s, idx_d, rows, acc, gsem, ssem):
    c = lax.axis_index("c")
    s = lax.axis_index("s")
    _prologue(src_hbm, dst_hbm, zeros_hbm, idx_s, idx_d, acc, gsem, ssem,
              c * NS + s, s)
    _pipelined_pass(wl_hbm, idx_s, idx_d, rows, acc, gsem, ssem)
    pltpu.sync_copy(acc.at[pl.ds(s * RPT, RPT)],
                    out_hbm.at[0, c, pl.ds(s * RPT, RPT)])
    pltpu.sync_copy(zeros_hbm.at[pl.ds(s * RPT, RPT)],
                    acc.at[pl.ds(s * RPT, RPT)])
    plsc.subcore_barrier()
    _pipelined_pass(wr_hbm, idx_s, idx_d, rows, acc, gsem, ssem)
    pltpu.sync_copy(acc.at[pl.ds(s * RPT, RPT)],
                    out_hbm.at[1, c, pl.ds(s * RPT, RPT)])

  return prop2


@functools.lru_cache(maxsize=None)
def _make_deg():

  @functools.partial(
      pl.kernel,
      out_type=jax.ShapeDtypeStruct((NC, NPAD, 16), jnp.float32),
      mesh=_mesh(),
      compiler_params=pltpu.CompilerParams(use_tc_tiling_on_sc=False),
      scratch_types=[
          pltpu.VMEM((NCHUNK, CH), jnp.int32),
          pltpu.VMEM((CH, 16), jnp.float32),
          pltpu.VMEM_SHARED((NPAD, 16), jnp.float32),
          pltpu.SemaphoreType.DMA((U,)),
      ],
  )
  def deg_kernel(ones_hbm, dst_hbm, zeros_hbm, out_hbm, idx_d, ones_v, acc,
                 ssem):
    c = lax.axis_index("c")
    s = lax.axis_index("s")
    wid = c * NS + s
    cps = [
        pltpu.async_copy(dst_hbm.at[wid], idx_d, ssem.at[0]),
        pltpu.async_copy(ones_hbm, ones_v, ssem.at[1]),
        pltpu.async_copy(zeros_hbm.at[pl.ds(s * RPT, RPT)],
                         acc.at[pl.ds(s * RPT, RPT)], ssem.at[2]),
    ]
    for cp in cps:
      cp.wait()
    plsc.subcore_barrier()

    def body(i, carry):
      base = i * U
      for j in range(U):
        @pl.when(i > 0)
        def _drain(j=j):
          pltpu.make_async_copy(
              ones_v, acc.at[idx_d.at[base - U + j]], ssem.at[j]).wait()
        pltpu.async_copy(ones_v, acc.at[idx_d.at[base + j]], ssem.at[j],
                         add=True)
      return carry

    lax.fori_loop(0, NG, body, 0)
    for j in range(U):
      pltpu.make_async_copy(
          ones_v, acc.at[idx_d.at[(NG - 1) * U + j]], ssem.at[j]).wait()
    plsc.subcore_barrier()
    pltpu.sync_copy(acc.at[pl.ds(s * RPT, RPT)],
                    out_hbm.at[c, pl.ds(s * RPT, RPT)])

  return deg_kernel


R = 2000
G = N // R
_P = jax.lax.Precision.HIGHEST


def _tc_call(body, in_specs, out_specs, out_shapes):
  return pl.pallas_call(
      body,
      grid=(G,),
      in_specs=in_specs,
      out_specs=out_specs,
      out_shape=out_shapes,
  )


def _b2(shape):
  nd = len(shape)
  return pl.BlockSpec(shape, lambda i: (0,) * nd)


_vp64 = pl.BlockSpec((NC, R, 64), lambda i: (0, i, 0))
_vp2 = pl.BlockSpec((2, NC, R, 64), lambda i: (0, 0, i, 0))
_n128 = pl.BlockSpec((R, 128), lambda i: (i, 0))
_n64 = pl.BlockSpec((R, 64), lambda i: (i, 0))
_n16 = pl.BlockSpec((R, 16), lambda i: (i, 0))


def _prep_body(degp, x, w10, acc1, wl, wr, dinv, dinv2):
  deg = degp[0, :, :] + degp[1, :, :]
  di = jnp.where(deg > 0.0, lax.rsqrt(jnp.maximum(deg, 1e-30)), 0.0)
  dinv[...] = di
  dinv2[...] = di * di
  xb = x[...]
  acc1[...] = jnp.dot(xb, w10[...], precision=_P)
  w = xb * di[:, 0:1]
  wl[...] = w[:, :64]
  wr[...] = w[:, 64:]


def _step1_body(vp2, dinv, dinv2, acc_in, wk, acc_out, wl, wr):
  v = jnp.concatenate([vp2[0, 0, :, :] + vp2[0, 1, :, :],
                       vp2[1, 0, :, :] + vp2[1, 1, :, :]], axis=1)
  di = dinv[:, 0:1]
  acc_out[...] = acc_in[...] + jnp.dot(v * di, wk[...], precision=_P)
  w = v * dinv2[:, 0:1]
  wl[...] = w[:, :64]
  wr[...] = w[:, 64:]


def _l1fin_body(vp2, dinv, acc_in, w13, b1, w20, w21, w22, w23,
                g0, g1, g2, w):
  v = jnp.concatenate([vp2[0, 0, :, :] + vp2[0, 1, :, :],
                       vp2[1, 0, :, :] + vp2[1, 1, :, :]], axis=1)
  di = dinv[:, 0:1]
  h = acc_in[...] + jnp.dot(v * di, w13[...], precision=_P) + b1[...]
  h = jnp.maximum(h, 0.0)
  g0[...] = jnp.dot(h, w20[...], precision=_P)
  g1[...] = jnp.dot(h, w21[...], precision=_P)
  g2[...] = jnp.dot(h, w22[...], precision=_P)
  w[...] = jnp.dot(h, w23[...], precision=_P) * di


def _step2_body(vp, dinv, dinv2, gk, w_next):
  v = vp[0, :, :] + vp[1, :, :]
  w_next[...] = gk[...] * dinv[:, 0:1] + v * dinv2[:, 0:1]


def _fin_body(vp, dinv, g0, b2, out):
  v = vp[0, :, :] + vp[1, :, :]
  t = g0[...] + v * dinv[:, 0:1] + b2[...]
  t = t - jnp.max(t, axis=1, keepdims=True)
  out[...] = t - jnp.log(jnp.sum(jnp.exp(t), axis=1, keepdims=True))


def kernel(x, edge_index, W1, b1, W2, b2):
  f32 = jnp.float32
  src3 = edge_index[0].reshape(NW, NCHUNK, CH)
  dst3 = edge_index[1].reshape(NW, NCHUNK, CH)
  z64 = jnp.zeros((NPAD, 64), f32)
  z16 = jnp.zeros((NPAD, 16), f32)
  ones16 = jnp.ones((CH, 16), f32)
  b1r = b1.reshape(1, 128)
  b2r = b2.reshape(1, 64)

  nshape128 = jax.ShapeDtypeStruct((N, 128), f32)
  nshape64 = jax.ShapeDtypeStruct((N, 64), f32)
  nshape16 = jax.ShapeDtypeStruct((N, 16), f32)

  deg_kernel = _make_deg()
  prop = _make_prop()
  prop2 = _make_prop2()

  degp = deg_kernel(ones16, dst3, z16)

  acc1, wl, wr, dinv, dinv2 = _tc_call(
      _prep_body,
      [pl.BlockSpec((NC, R, 16), lambda i: (0, i, 0)), _n128, _b2((128, 128))],
      [_n128, _n64, _n64, _n16, _n16],
      [nshape128, nshape64, nshape64, nshape16, nshape16],
  )(degp, x, W1[0])

  for k in (1, 2):
    vp2 = prop2(wl, wr, src3, dst3, z64)
    acc1, wl, wr = _tc_call(
        _step1_body,
        [_vp2, _n16, _n16, _n128, _b2((128, 128))],
        [_n128, _n64, _n64],
        [nshape128, nshape64, nshape64],
    )(vp2, dinv, dinv2, acc1, W1[k])

  vp2 = prop2(wl, wr, src3, dst3, z64)
  g0, g1, g2, w = _tc_call(
      _l1fin_body,
      [_vp2, _n16, _n128, _b2((128, 128)), _b2((1, 128)),
       _b2((128, 64)), _b2((128, 64)), _b2((128, 64)), _b2((128, 64))],
      [_n64, _n64, _n64, _n64],
      [nshape64, nshape64, nshape64, nshape64],
  )(vp2, dinv, acc1, W1[3], b1r, W2[0], W2[1], W2[2], W2[3])

  for gk in (g2, g1):
    vp = prop(w, src3, dst3, z64)
    (w,) = _tc_call(
        _step2_body,
        [_vp64, _n16, _n16, _n64],
        [_n64],
        [nshape64],
    )(vp, dinv, dinv2, gk)

  vp = prop(w, src3, dst3, z64)
  (out,) = _tc_call(
      _fin_body,
      [_vp64, _n16, _n64, _b2((1, 64))],
      [_n64],
      [nshape64],
  )(vp, dinv, g0, b2r)
  return out
